# Initial kernel scaffold; baseline (speedup 1.0000x reference)
import jax
import jax.numpy as jnp
from jax.experimental import pallas as pl


def kernel(x0, x1, x2, x3, edge_index0, edge_index1, edge_index2, edge_index3, x_node, num_node, W_g, b_g, W_d, b_d, W_c, b_c, W_s, b_s, u, W_lin, b_lin):
    raise NotImplementedError("write your pallas kernel here")


# speedup vs baseline: 5.9907x; 5.9907x over previous
import functools

import jax
import jax.numpy as jnp
from jax import lax
from jax.experimental import pallas as pl
from jax.experimental.pallas import tpu as pltpu
from jax.experimental.pallas import tpu_sc as plsc

N = 10000
E = 320000
D = 128

CH = 125
NCORES = 2
NSUB = 16
ROWS_PER_TILE = E // CH // (NCORES * NSUB)
NPAD = 10112
NPT = NPAD // NSUB

BN = 1000


def _dense_pre_body(x0, x1, x2, x3, w, b, o0, o1, o2, o3):
    ws = w[...]
    bs = b[...]
    for r, (xr, orr) in enumerate(((x0, o0), (x1, o1), (x2, o2), (x3, o3))):
        h = jnp.dot(xr[...], ws[r], preferred_element_type=jnp.float32)
        orr[...] = jnp.maximum(h + bs[r, 0][None, :], 0.0)


def _dense_pre(xs, Wstk, bstk):
    grid = (N // BN,)
    xspec = pl.BlockSpec((BN, D), lambda i: (i, 0))
    return pl.pallas_call(
        _dense_pre_body,
        grid=grid,
        in_specs=[xspec] * 4 + [
            pl.BlockSpec((4, D, D), lambda i: (0, 0, 0)),
            pl.BlockSpec((4, 1, D), lambda i: (0, 0, 0)),
        ],
        out_specs=[xspec] * 4,
        out_shape=[jax.ShapeDtypeStruct((N, D), jnp.float32)] * 4,
    )(*xs, Wstk, bstk)


def _sc_agg_body(xt0, xt1, xt2, xt3, src0, src1, src2, src3,
                 tgt0, tgt1, tgt2, tgt3, z128, z16, ones_h,
                 a0, a1, a2, a3, c0, c1, c2, c3,
                 aggr_s, cnt_s, src_v, tgt_v, rows_v, ones_v, sem):
    c = lax.axis_index("c")
    s = lax.axis_index("s")
    row0 = s * NPT

    pltpu.sync_copy(ones_h, ones_v)

    xts = (xt0, xt1, xt2, xt3)
    srcs = (src0, src1, src2, src3)
    tgts = (tgt0, tgt1, tgt2, tgt3)
    aouts = (a0, a1, a2, a3)
    couts = (c0, c1, c2, c3)

    base = c * (NSUB * ROWS_PER_TILE) + s * ROWS_PER_TILE

    for r in range(4):
        pltpu.sync_copy(z128.at[pl.ds(row0, NPT)], aggr_s.at[pl.ds(row0, NPT)])
        pltpu.sync_copy(z16.at[pl.ds(row0, NPT)], cnt_s.at[pl.ds(row0, NPT)])
        pltpu.sync_copy(srcs[r].at[pl.ds(base, ROWS_PER_TILE)], src_v)
        pltpu.sync_copy(tgts[r].at[pl.ds(base, ROWS_PER_TILE)], tgt_v)
        plsc.subcore_barrier()

        def body(j, carry):
            pltpu.async_copy(xts[r].at[tgt_v.at[j]], rows_v, sem).wait()
            pltpu.sync_copy(rows_v, aggr_s.at[src_v.at[j]], add=True)
            pltpu.sync_copy(ones_v, cnt_s.at[src_v.at[j]], add=True)
            return carry

        lax.fori_loop(0, ROWS_PER_TILE, body, 0)
        plsc.subcore_barrier()
        pltpu.sync_copy(aggr_s.at[pl.ds(row0, NPT)],
                        aouts[r].at[c, pl.ds(row0, NPT)])
        pltpu.sync_copy(cnt_s.at[pl.ds(row0, NPT)],
                        couts[r].at[c, pl.ds(row0, NPT)])
        plsc.subcore_barrier()


def _sc_agg(xts, srcs, tgts):
    mesh = plsc.VectorSubcoreMesh(core_axis_name="c", subcore_axis_name="s")
    z128 = jnp.zeros((NPAD, D), jnp.float32)
    z16 = jnp.zeros((NPAD, 16), jnp.float32)
    ones_h = jnp.ones((CH, 16), jnp.float32)
    out_type = ([jax.ShapeDtypeStruct((NCORES, NPAD, D), jnp.float32)] * 4
                + [jax.ShapeDtypeStruct((NCORES, NPAD, 16), jnp.float32)] * 4)
    scratch = [
        pltpu.VMEM_SHARED((NPAD, D), jnp.float32),
        pltpu.VMEM_SHARED((NPAD, 16), jnp.float32),
        pltpu.VMEM((ROWS_PER_TILE, CH), jnp.int32),
        pltpu.VMEM((ROWS_PER_TILE, CH), jnp.int32),
        pltpu.VMEM((CH, D), jnp.float32),
        pltpu.VMEM((CH, 16), jnp.float32),
        pltpu.SemaphoreType.DMA,
    ]
    fn = pl.kernel(_sc_agg_body, out_type=out_type, mesh=mesh,
                   scratch_types=scratch,
                   compiler_params=pltpu.CompilerParams(
                       use_tc_tiling_on_sc=False))
    return fn(*xts, *srcs, *tgts, z128, z16, ones_h)


def _combine_body(p0, p1, p2, p3, q0, q1, q2, q3, xn_ref, u_ref, wl_ref,
                  bl_ref, out_ref):
    xn = xn_ref[...]
    uu = u_ref[...]
    u_top = uu[0, :D]
    u_bot = uu[0, D:]
    xn_dot = jnp.sum(xn * u_bot[None, :], axis=1)

    means = []
    scores = []
    for p_ref, q_ref in ((p0, q0), (p1, q1), (p2, q2), (p3, q3)):
        p = p_ref[0] + p_ref[1]
        q = q_ref[0] + q_ref[1]
        cnt = jnp.maximum(q[:, 0], 1.0)
        mean = p / cnt[:, None]
        e = jnp.sum(mean * u_top[None, :], axis=1) + xn_dot
        e = jnp.where(e >= 0.0, e, 0.01 * e)
        scores.append(jnp.exp(e))
        means.append(mean)

    tot = scores[0] + scores[1] + scores[2] + scores[3]
    combined = ((scores[0] / tot)[:, None] * means[0]
                + (scores[1] / tot)[:, None] * means[1]
                + (scores[2] / tot)[:, None] * means[2]
                + (scores[3] / tot)[:, None] * means[3])

    wl = wl_ref[...]
    out = (jnp.dot(xn, wl[:D], preferred_element_type=jnp.float32)
           + jnp.dot(combined, wl[D:], preferred_element_type=jnp.float32)
           + bl_ref[0][None, :])
    out = jnp.maximum(out, 0.0)
    nrm = jnp.maximum(jnp.sqrt(jnp.sum(out * out, axis=1)), 1e-12)
    out_ref[...] = out / nrm[:, None]


def _combine(parts, cnts, x_node, u, W_lin, b_lin):
    grid = (N // BN,)
    pspec = pl.BlockSpec((NCORES, BN, D), lambda i: (0, i, 0))
    qspec = pl.BlockSpec((NCORES, BN, 16), lambda i: (0, i, 0))
    xspec = pl.BlockSpec((BN, D), lambda i: (i, 0))
    return pl.pallas_call(
        _combine_body,
        grid=grid,
        in_specs=[pspec] * 4 + [qspec] * 4 + [
            xspec,
            pl.BlockSpec((1, 2 * D), lambda i: (0, 0)),
            pl.BlockSpec((2 * D, D), lambda i: (0, 0)),
            pl.BlockSpec((1, D), lambda i: (0, 0)),
        ],
        out_specs=xspec,
        out_shape=jax.ShapeDtypeStruct((N, D), jnp.float32),
    )(*parts, *cnts, x_node, u, W_lin, b_lin)


def kernel(x0, x1, x2, x3, edge_index0, edge_index1, edge_index2, edge_index3,
           x_node, num_node, W_g, b_g, W_d, b_d, W_c, b_c, W_s, b_s, u,
           W_lin, b_lin):
    Wstk = jnp.stack((W_g, W_d, W_c, W_s))
    bstk = jnp.stack((b_g, b_d, b_c, b_s)).reshape(4, 1, D)
    xts = _dense_pre((x0, x1, x2, x3), Wstk, bstk)

    edges = (edge_index0, edge_index1, edge_index2, edge_index3)
    srcs = [e[0].reshape(E // CH, CH) for e in edges]
    tgts = [e[1].reshape(E // CH, CH) for e in edges]
    outs = _sc_agg(xts, srcs, tgts)
    parts, cnts = outs[:4], outs[4:]

    return _combine(parts, cnts, x_node, u.reshape(1, 2 * D), W_lin,
                    b_lin.reshape(1, D))


# speedup vs baseline: 7.3448x; 1.2260x over previous
import functools

import jax
import jax.numpy as jnp
from jax import lax
from jax.experimental import pallas as pl
from jax.experimental.pallas import tpu as pltpu
from jax.experimental.pallas import tpu_sc as plsc

N = 10000
E = 320000
D = 128
DW = 144

CH = 125
NCORES = 2
NSUB = 16
ROWS_PER_TILE = E // CH // (NCORES * NSUB)
NPAD = 10112
NPT = NPAD // NSUB

BN = 1000


def _dense_pre_body(x0, x1, x2, x3, w, b, o0, o1, o2, o3):
    ws = w[...]
    bs = b[...]
    ones = jnp.ones((BN, DW - D), jnp.float32)
    for r, (xr, orr) in enumerate(((x0, o0), (x1, o1), (x2, o2), (x3, o3))):
        h = jnp.dot(xr[...], ws[r], preferred_element_type=jnp.float32)
        h = jnp.maximum(h + bs[r, 0][None, :], 0.0)
        orr[...] = jnp.concatenate((h, ones), axis=-1)


def _dense_pre(xs, Wstk, bstk):
    grid = (N // BN,)
    xspec = pl.BlockSpec((BN, D), lambda i: (i, 0))
    ospec = pl.BlockSpec((BN, DW), lambda i: (i, 0))
    return pl.pallas_call(
        _dense_pre_body,
        grid=grid,
        in_specs=[xspec] * 4 + [
            pl.BlockSpec((4, D, D), lambda i: (0, 0, 0)),
            pl.BlockSpec((4, 1, D), lambda i: (0, 0, 0)),
        ],
        out_specs=[ospec] * 4,
        out_shape=[jax.ShapeDtypeStruct((N, DW), jnp.float32)] * 4,
    )(*xs, Wstk, bstk)


PAIRS_PER_TILE = ROWS_PER_TILE // 2


def _sc_agg_body(xt0, xt1, xt2, xt3, eix0, eix1, eix2, eix3, zeros_h,
                 a0, a1, a2, a3,
                 aggr_s, tix_v, rows_v, semg0, semg1, semi):
    c = lax.axis_index("c")
    s = lax.axis_index("s")
    row0 = s * NPT

    xts = (xt0, xt1, xt2, xt3)
    eixs = (eix0, eix1, eix2, eix3)
    aouts = (a0, a1, a2, a3)
    semg = (semg0, semg1)

    pbase = (c * NSUB + s) * PAIRS_PER_TILE

    for r in range(4):
        pltpu.sync_copy(zeros_h.at[pl.ds(row0, NPT)],
                        aggr_s.at[pl.ds(row0, NPT)])
        plsc.subcore_barrier()

        pltpu.sync_copy(eixs[r].at[pbase], tix_v.at[0])
        pltpu.async_copy(eixs[r].at[pbase + 1], tix_v.at[1], semi)
        for b in range(2):
            pltpu.async_copy(xts[r].at[tix_v.at[0, b, 0]], rows_v.at[b],
                             semg[b])

        def step(p, carry):
            q = lax.rem(p, 2)
            qn = lax.rem(p + 1, 2)
            for b in range(2):
                pltpu.make_async_copy(xts[r].at[tix_v.at[q, b, 0]],
                                      rows_v.at[b], semg[b]).wait()
                pltpu.sync_copy(rows_v.at[b], aggr_s.at[tix_v.at[q, b, 1]],
                                add=True)

                @pl.when(p + 1 < PAIRS_PER_TILE)
                def _():
                    if b == 0:
                        pltpu.make_async_copy(eixs[r].at[pbase + p + 1],
                                              tix_v.at[qn], semi).wait()
                    pltpu.async_copy(xts[r].at[tix_v.at[qn, b, 0]],
                                     rows_v.at[b], semg[b])

            @pl.when(p + 2 < PAIRS_PER_TILE)
            def _():
                pltpu.async_copy(eixs[r].at[pbase + p + 2], tix_v.at[q], semi)

            return carry

        lax.fori_loop(0, PAIRS_PER_TILE, step, 0)
        plsc.subcore_barrier()
        pltpu.sync_copy(aggr_s.at[pl.ds(row0, NPT)],
                        aouts[r].at[c, pl.ds(row0, NPT)])
        plsc.subcore_barrier()


def _sc_agg(xts, eixs):
    mesh = plsc.VectorSubcoreMesh(core_axis_name="c", subcore_axis_name="s")
    zeros_h = jnp.zeros((NPAD, DW), jnp.float32)
    out_type = [jax.ShapeDtypeStruct((NCORES, NPAD, DW), jnp.float32)] * 4
    scratch = [
        pltpu.VMEM_SHARED((NPAD, DW), jnp.float32),
        pltpu.VMEM((2, 2, 2, CH), jnp.int32),
        pltpu.VMEM((2, CH, DW), jnp.float32),
        pltpu.SemaphoreType.DMA,
        pltpu.SemaphoreType.DMA,
        pltpu.SemaphoreType.DMA,
    ]
    fn = pl.kernel(_sc_agg_body, out_type=out_type, mesh=mesh,
                   scratch_types=scratch,
                   compiler_params=pltpu.CompilerParams(
                       use_tc_tiling_on_sc=False))
    return fn(*xts, *eixs, zeros_h)


def _combine_body(p0, p1, p2, p3, xn_ref, u_ref, wl_ref, bl_ref, out_ref):
    xn = xn_ref[...]
    uu = u_ref[...]
    u_top = uu[0, :D]
    u_bot = uu[0, D:]
    xn_dot = jnp.sum(xn * u_bot[None, :], axis=1)

    means = []
    scores = []
    for p_ref in (p0, p1, p2, p3):
        p = p_ref[0] + p_ref[1]
        cnt = jnp.maximum(p[:, D], 1.0)
        mean = p[:, :D] / cnt[:, None]
        e = jnp.sum(mean * u_top[None, :], axis=1) + xn_dot
        e = jnp.where(e >= 0.0, e, 0.01 * e)
        scores.append(jnp.exp(e))
        means.append(mean)

    tot = scores[0] + scores[1] + scores[2] + scores[3]
    combined = ((scores[0] / tot)[:, None] * means[0]
                + (scores[1] / tot)[:, None] * means[1]
                + (scores[2] / tot)[:, None] * means[2]
                + (scores[3] / tot)[:, None] * means[3])

    wl = wl_ref[...]
    out = (jnp.dot(xn, wl[:D], preferred_element_type=jnp.float32)
           + jnp.dot(combined, wl[D:], preferred_element_type=jnp.float32)
           + bl_ref[0][None, :])
    out = jnp.maximum(out, 0.0)
    nrm = jnp.maximum(jnp.sqrt(jnp.sum(out * out, axis=1)), 1e-12)
    out_ref[...] = out / nrm[:, None]


def _combine(parts, x_node, u, W_lin, b_lin):
    grid = (N // BN,)
    pspec = pl.BlockSpec((NCORES, BN, DW), lambda i: (0, i, 0))
    xspec = pl.BlockSpec((BN, D), lambda i: (i, 0))
    return pl.pallas_call(
        _combine_body,
        grid=grid,
        in_specs=[pspec] * 4 + [
            xspec,
            pl.BlockSpec((1, 2 * D), lambda i: (0, 0)),
            pl.BlockSpec((2 * D, D), lambda i: (0, 0)),
            pl.BlockSpec((1, D), lambda i: (0, 0)),
        ],
        out_specs=xspec,
        out_shape=jax.ShapeDtypeStruct((N, D), jnp.float32),
    )(*parts, x_node, u, W_lin, b_lin)


def kernel(x0, x1, x2, x3, edge_index0, edge_index1, edge_index2, edge_index3,
           x_node, num_node, W_g, b_g, W_d, b_d, W_c, b_c, W_s, b_s, u,
           W_lin, b_lin):
    Wstk = jnp.stack((W_g, W_d, W_c, W_s))
    bstk = jnp.stack((b_g, b_d, b_c, b_s)).reshape(4, 1, D)
    xts = _dense_pre((x0, x1, x2, x3), Wstk, bstk)

    edges = (edge_index0, edge_index1, edge_index2, edge_index3)
    eixs = [jnp.stack((e[1].reshape(E // CH // 2, 2, CH),
                       e[0].reshape(E // CH // 2, 2, CH)), axis=2)
            for e in edges]
    parts = _sc_agg(xts, eixs)

    return _combine(parts, x_node, u.reshape(1, 2 * D), W_lin,
                    b_lin.reshape(1, D))


# speedup vs baseline: 8.7137x; 1.1864x over previous
import functools

import jax
import jax.numpy as jnp
from jax import lax
from jax.experimental import pallas as pl
from jax.experimental.pallas import tpu as pltpu
from jax.experimental.pallas import tpu_sc as plsc

N = 10000
E = 320000
D = 128
DW = 144

CH = 125
NCORES = 2
NSUB = 16
ROWS_PER_TILE = E // CH // (NCORES * NSUB)
NPAD = 10112
NPT = NPAD // NSUB

BN = 1000


def _dense_pre_body(x0, x1, x2, x3, w, b, o0, o1, o2, o3):
    ws = w[...]
    bs = b[...]
    ones = jnp.ones((BN, DW - D), jnp.float32)
    for r, (xr, orr) in enumerate(((x0, o0), (x1, o1), (x2, o2), (x3, o3))):
        h = jnp.dot(xr[...], ws[r], preferred_element_type=jnp.float32)
        h = jnp.maximum(h + bs[r, 0][None, :], 0.0)
        orr[...] = jnp.concatenate((h, ones), axis=-1)


def _dense_pre(xs, Wstk, bstk):
    grid = (N // BN,)
    xspec = pl.BlockSpec((BN, D), lambda i: (i, 0))
    ospec = pl.BlockSpec((BN, DW), lambda i: (i, 0))
    return pl.pallas_call(
        _dense_pre_body,
        grid=grid,
        in_specs=[xspec] * 4 + [
            pl.BlockSpec((4, D, D), lambda i: (0, 0, 0)),
            pl.BlockSpec((4, 1, D), lambda i: (0, 0, 0)),
        ],
        out_specs=[ospec] * 4,
        out_shape=[jax.ShapeDtypeStruct((N, DW), jnp.float32)] * 4,
    )(*xs, Wstk, bstk)


PAIRS_PER_TILE = E // CH // 2 // NSUB


def _relation_pipe(r, xt, tgt2, src2, sums_out, cnt_out, zeros_h,
                   aggr_s, tix_v, rows_v, semg, semi, s, row0):
    pbase = s * PAIRS_PER_TILE

    pltpu.sync_copy(zeros_h.at[pl.ds(row0, NPT)], aggr_s.at[pl.ds(row0, NPT)])
    plsc.subcore_barrier()

    pltpu.sync_copy(tgt2.at[pbase], tix_v.at[0, 0])
    pltpu.sync_copy(src2.at[pbase], tix_v.at[0, 1])
    pltpu.async_copy(tgt2.at[pbase + 1], tix_v.at[1, 0], semi)
    pltpu.async_copy(src2.at[pbase + 1], tix_v.at[1, 1], semi)
    for b in range(2):
        pltpu.async_copy(xt.at[tix_v.at[0, 0, b]], rows_v.at[b], semg[b])

    def step(p, carry):
        q = lax.rem(p, 2)
        qn = lax.rem(p + 1, 2)
        for b in range(2):
            pltpu.make_async_copy(xt.at[tix_v.at[q, 0, b]],
                                  rows_v.at[b], semg[b]).wait()
            pltpu.sync_copy(rows_v.at[b], aggr_s.at[tix_v.at[q, 1, b]],
                            add=True)

            @pl.when(p + 1 < PAIRS_PER_TILE)
            def _():
                if b == 0:
                    pltpu.make_async_copy(tgt2.at[pbase + p + 1],
                                          tix_v.at[qn, 0], semi).wait()
                    pltpu.make_async_copy(src2.at[pbase + p + 1],
                                          tix_v.at[qn, 1], semi).wait()
                pltpu.async_copy(xt.at[tix_v.at[qn, 0, b]], rows_v.at[b],
                                 semg[b])

        @pl.when(p + 2 < PAIRS_PER_TILE)
        def _():
            pltpu.async_copy(tgt2.at[pbase + p + 2], tix_v.at[q, 0], semi)
            pltpu.async_copy(src2.at[pbase + p + 2], tix_v.at[q, 1], semi)

        return carry

    lax.fori_loop(0, PAIRS_PER_TILE, step, 0)
    plsc.subcore_barrier()
    pltpu.sync_copy(aggr_s.at[pl.ds(row0, NPT), pl.ds(0, D)],
                    sums_out.at[pl.ds(row0, NPT)])
    pltpu.sync_copy(aggr_s.at[pl.ds(row0, NPT), pl.ds(D, 16)],
                    cnt_out.at[pl.ds(row0, NPT), pl.ds(16 * r, 16)])
    plsc.subcore_barrier()


def _sc_agg_body(xt0, xt1, xt2, xt3, tgt0, tgt1, tgt2, tgt3,
                 src0, src1, src2, src3, zeros_h,
                 a0, a1, a2, a3, cnt_out,
                 aggr_s, tix_v, rows_v, semg0, semg1, semi):
    c = lax.axis_index("c")
    s = lax.axis_index("s")
    row0 = s * NPT

    xts = (xt0, xt1, xt2, xt3)
    tgts = (tgt0, tgt1, tgt2, tgt3)
    srcs = (src0, src1, src2, src3)
    aouts = (a0, a1, a2, a3)
    semg = (semg0, semg1)

    for core in range(NCORES):
        @pl.when(c == core)
        def _():
            for r in (2 * core, 2 * core + 1):
                _relation_pipe(r, xts[r], tgts[r], srcs[r], aouts[r],
                               cnt_out, zeros_h, aggr_s, tix_v, rows_v,
                               semg, semi, s, row0)


def _sc_agg(xts, tgts, srcs):
    mesh = plsc.VectorSubcoreMesh(core_axis_name="c", subcore_axis_name="s")
    zeros_h = jnp.zeros((NPAD, DW), jnp.float32)
    out_type = ([jax.ShapeDtypeStruct((NPAD, D), jnp.float32)] * 4
                + [jax.ShapeDtypeStruct((NPAD, 64), jnp.float32)])
    scratch = [
        pltpu.VMEM_SHARED((NPAD, DW), jnp.float32),
        pltpu.VMEM((2, 2, 2, CH), jnp.int32),
        pltpu.VMEM((2, CH, DW), jnp.float32),
        pltpu.SemaphoreType.DMA,
        pltpu.SemaphoreType.DMA,
        pltpu.SemaphoreType.DMA,
    ]
    fn = pl.kernel(_sc_agg_body, out_type=out_type, mesh=mesh,
                   scratch_types=scratch,
                   compiler_params=pltpu.CompilerParams(
                       use_tc_tiling_on_sc=False))
    return fn(*xts, *tgts, *srcs, zeros_h)


def _combine_body(p0, p1, p2, p3, q_ref, xn_ref, u_ref, wl_ref, bl_ref,
                  out_ref):
    xn = xn_ref[...]
    uu = u_ref[...]
    u_top = uu[0, :D]
    u_bot = uu[0, D:]
    xn_dot = jnp.sum(xn * u_bot[None, :], axis=1)
    q = q_ref[...]

    means = []
    scores = []
    for r, p_ref in enumerate((p0, p1, p2, p3)):
        cnt = jnp.maximum(q[:, 16 * r], 1.0)
        mean = p_ref[...] / cnt[:, None]
        e = jnp.sum(mean * u_top[None, :], axis=1) + xn_dot
        e = jnp.where(e >= 0.0, e, 0.01 * e)
        scores.append(jnp.exp(e))
        means.append(mean)

    tot = scores[0] + scores[1] + scores[2] + scores[3]
    combined = ((scores[0] / tot)[:, None] * means[0]
                + (scores[1] / tot)[:, None] * means[1]
                + (scores[2] / tot)[:, None] * means[2]
                + (scores[3] / tot)[:, None] * means[3])

    wl = wl_ref[...]
    out = (jnp.dot(xn, wl[:D], preferred_element_type=jnp.float32)
           + jnp.dot(combined, wl[D:], preferred_element_type=jnp.float32)
           + bl_ref[0][None, :])
    out = jnp.maximum(out, 0.0)
    nrm = jnp.maximum(jnp.sqrt(jnp.sum(out * out, axis=1)), 1e-12)
    out_ref[...] = out / nrm[:, None]


def _combine(parts, cnts, x_node, u, W_lin, b_lin):
    grid = (N // BN,)
    pspec = pl.BlockSpec((BN, D), lambda i: (i, 0))
    qspec = pl.BlockSpec((BN, 64), lambda i: (i, 0))
    xspec = pl.BlockSpec((BN, D), lambda i: (i, 0))
    return pl.pallas_call(
        _combine_body,
        grid=grid,
        in_specs=[pspec] * 4 + [qspec] + [
            xspec,
            pl.BlockSpec((1, 2 * D), lambda i: (0, 0)),
            pl.BlockSpec((2 * D, D), lambda i: (0, 0)),
            pl.BlockSpec((1, D), lambda i: (0, 0)),
        ],
        out_specs=xspec,
        out_shape=jax.ShapeDtypeStruct((N, D), jnp.float32),
    )(*parts, cnts, x_node, u, W_lin, b_lin)


def kernel(x0, x1, x2, x3, edge_index0, edge_index1, edge_index2, edge_index3,
           x_node, num_node, W_g, b_g, W_d, b_d, W_c, b_c, W_s, b_s, u,
           W_lin, b_lin):
    Wstk = jnp.stack((W_g, W_d, W_c, W_s))
    bstk = jnp.stack((b_g, b_d, b_c, b_s)).reshape(4, 1, D)
    xts = _dense_pre((x0, x1, x2, x3), Wstk, bstk)

    edges = (edge_index0, edge_index1, edge_index2, edge_index3)
    tgts = [e[1].reshape(E // CH // 2, 2, CH) for e in edges]
    srcs = [e[0].reshape(E // CH // 2, 2, CH) for e in edges]
    outs = _sc_agg(xts, tgts, srcs)

    return _combine(outs[:4], outs[4], x_node, u.reshape(1, 2 * D), W_lin,
                    b_lin.reshape(1, D))
